# Initial kernel scaffold; baseline (speedup 1.0000x reference)
import functools

import jax
import jax.numpy as jnp
from jax import lax
from jax.experimental import pallas as pl
from jax.experimental.pallas import tpu as pltpu
from jax.experimental.pallas import tpu_sc as plsc

_FEAT = 32
_B = 16384 * 50
_NW = 32
_BPW = _B // _NW
_CHUNK = 1280
_NCHUNK = _BPW // _CHUNK


def _embed_body(table_hbm, idx_hbm, out_hbm, idx_v, rows0, rows1, sem0, sem1):
    wid = lax.axis_index("s") * 2 + lax.axis_index("c")
    base = wid * _BPW
    pltpu.sync_copy(idx_hbm.at[pl.ds(base, _BPW)], idx_v)

    bufs = (rows0, rows1)
    sems = (sem0, sem1)
    copies = [None, None]
    copies[0] = pltpu.async_copy(
        table_hbm.at[idx_v.at[pl.ds(0, _CHUNK)]], bufs[0], sems[0])
    for g in range(_NCHUNK):
        if g + 1 < _NCHUNK:
            copies[(g + 1) % 2] = pltpu.async_copy(
                table_hbm.at[idx_v.at[pl.ds((g + 1) * _CHUNK, _CHUNK)]],
                bufs[(g + 1) % 2], sems[(g + 1) % 2])
        copies[g % 2].wait()
        pltpu.sync_copy(bufs[g % 2],
                        out_hbm.at[pl.ds(base + g * _CHUNK, _CHUNK)])


@functools.partial(
    pl.kernel,
    mesh=plsc.VectorSubcoreMesh(core_axis_name="c", subcore_axis_name="s"),
    out_type=jax.ShapeDtypeStruct((_B, _FEAT), jnp.float32),
    scratch_types=[
        pltpu.VMEM((_BPW,), jnp.int32),
        pltpu.VMEM((_CHUNK, _FEAT), jnp.float32),
        pltpu.VMEM((_CHUNK, _FEAT), jnp.float32),
        pltpu.SemaphoreType.DMA,
        pltpu.SemaphoreType.DMA,
    ],
)
def _embed_gather(table_hbm, idx_hbm, out_hbm, idx_v, rows0, rows1, sem0, sem1):
    _embed_body(table_hbm, idx_hbm, out_hbm, idx_v, rows0, rows1, sem0, sem1)


def kernel(metadata, embedding):
    m = metadata
    if m.ndim > 0 and m.shape[-1] == 1:
        m = jnp.squeeze(m, axis=-1)
    idx = m.reshape(-1)
    out = _embed_gather(embedding, idx)
    return out.reshape(m.shape + (embedding.shape[1],))


# speedup vs baseline: 1.1125x; 1.1125x over previous
import functools

import jax
import jax.numpy as jnp
from jax import lax
from jax.experimental import pallas as pl
from jax.experimental.pallas import tpu as pltpu
from jax.experimental.pallas import tpu_sc as plsc

_FEAT = 32
_B = 16384 * 50
_NW = 32
_BPW = _B // _NW
_CHUNK = 1280
_NCHUNK = _BPW // _CHUNK


def _embed_body(table_hbm, idx_hbm, out_hbm, idx_v, rows0, rows1, sem0, sem1):
    wid = lax.axis_index("s") * 2 + lax.axis_index("c")
    base = wid * _BPW
    pltpu.sync_copy(idx_hbm.at[pl.ds(base, _BPW)], idx_v)

    bufs = (rows0, rows1)
    sems = (sem0, sem1)
    copies = [None, None]
    copies[0] = pltpu.async_copy(
        table_hbm.at[idx_v.at[pl.ds(0, _CHUNK)]], bufs[0], sems[0])
    for g in range(_NCHUNK):
        if g + 1 < _NCHUNK:
            copies[(g + 1) % 2] = pltpu.async_copy(
                table_hbm.at[idx_v.at[pl.ds((g + 1) * _CHUNK, _CHUNK)]],
                bufs[(g + 1) % 2], sems[(g + 1) % 2])
        copies[g % 2].wait()
        pltpu.sync_copy(bufs[g % 2],
                        out_hbm.at[pl.ds(base + g * _CHUNK, _CHUNK)])


@functools.partial(
    pl.kernel,
    mesh=plsc.VectorSubcoreMesh(core_axis_name="c", subcore_axis_name="s"),
    out_type=jax.ShapeDtypeStruct((_B, _FEAT), jnp.float32),
    scratch_types=[
        pltpu.VMEM((_BPW,), jnp.int32),
        pltpu.VMEM((_CHUNK, _FEAT), jnp.float32),
        pltpu.VMEM((_CHUNK, _FEAT), jnp.float32),
        pltpu.SemaphoreType.DMA,
        pltpu.SemaphoreType.DMA,
    ],
    compiler_params=pltpu.CompilerParams(use_tc_tiling_on_sc=False),
)
def _embed_gather(table_hbm, idx_hbm, out_hbm, idx_v, rows0, rows1, sem0, sem1):
    _embed_body(table_hbm, idx_hbm, out_hbm, idx_v, rows0, rows1, sem0, sem1)


def kernel(metadata, embedding):
    m = metadata
    if m.ndim > 0 and m.shape[-1] == 1:
        m = jnp.squeeze(m, axis=-1)
    idx = m.reshape(-1)
    out = _embed_gather(embedding, idx)
    return out.reshape(m.shape + (embedding.shape[1],))


# speedup vs baseline: 1.1139x; 1.0012x over previous
import functools

import jax
import jax.numpy as jnp
from jax import lax
from jax.experimental import pallas as pl
from jax.experimental.pallas import tpu as pltpu
from jax.experimental.pallas import tpu_sc as plsc

_FEAT = 32
_B = 16384 * 50
_NW = 32
_BPW = _B // _NW
_NBUF = 4
_CHUNK = 800
_NCHUNK = _BPW // _CHUNK


def _embed_body(table_hbm, idx_hbm, out_hbm, idx_v, bufs, sems):
    wid = lax.axis_index("s") * 2 + lax.axis_index("c")
    base = wid * _BPW
    pltpu.sync_copy(idx_hbm.at[pl.ds(base, _BPW)], idx_v)

    def gather(chunk, b):
        return pltpu.async_copy(
            table_hbm.at[idx_v.at[pl.ds(chunk * _CHUNK, _CHUNK)]],
            bufs[b], sems[b])

    def wait_gather(b):
        pltpu.make_async_copy(
            table_hbm.at[idx_v.at[pl.ds(0, _CHUNK)]], bufs[b], sems[b]).wait()

    def store(chunk, b):
        pltpu.sync_copy(bufs[b], out_hbm.at[pl.ds(base + chunk * _CHUNK,
                                                  _CHUNK)])

    for b in range(_NBUF):
        gather(b, b)

    def body(i, _):
        g0 = i * _NBUF
        for b in range(_NBUF):
            g = g0 + b
            wait_gather(b)
            store(g, b)
            gather(g + _NBUF, b)
        return _

    lax.fori_loop(0, (_NCHUNK - _NBUF) // _NBUF, body, None)

    for b in range(_NBUF):
        g = _NCHUNK - _NBUF + b
        wait_gather(b)
        store(g, b)


@functools.partial(
    pl.kernel,
    mesh=plsc.VectorSubcoreMesh(core_axis_name="c", subcore_axis_name="s"),
    out_type=jax.ShapeDtypeStruct((_B, _FEAT), jnp.float32),
    scratch_types=[
        pltpu.VMEM((_BPW,), jnp.int32),
    ] + [pltpu.VMEM((_CHUNK, _FEAT), jnp.float32) for _ in range(_NBUF)]
      + [pltpu.SemaphoreType.DMA for _ in range(_NBUF)],
    compiler_params=pltpu.CompilerParams(use_tc_tiling_on_sc=False),
)
def _embed_gather(table_hbm, idx_hbm, out_hbm, idx_v, *rest):
    bufs = rest[:_NBUF]
    sems = rest[_NBUF:]
    _embed_body(table_hbm, idx_hbm, out_hbm, idx_v, bufs, sems)


def kernel(metadata, embedding):
    m = metadata
    if m.ndim > 0 and m.shape[-1] == 1:
        m = jnp.squeeze(m, axis=-1)
    idx = m.reshape(-1)
    out = _embed_gather(embedding, idx)
    return out.reshape(m.shape + (embedding.shape[1],))


# speedup vs baseline: 1.7941x; 1.6107x over previous
import functools

import jax
import jax.numpy as jnp
from jax import lax
from jax.experimental import pallas as pl
from jax.experimental.pallas import tpu as pltpu
from jax.experimental.pallas import tpu_sc as plsc

_FEAT = 32
_ROWS = 16384
_HIST = 50
_NW = 32
_RPW = _ROWS // _NW
_NBUF = 8


def _embed_body(table_hbm, idx_hbm, out_hbm, idx2d, bufs, sems):
    wid = lax.axis_index("s") * 2 + lax.axis_index("c")
    base = wid * _RPW
    pltpu.sync_copy(idx_hbm.at[pl.ds(base, _RPW)], idx2d)

    def gather(row, b):
        pltpu.async_copy(table_hbm.at[idx2d.at[row]], bufs[b], sems[b])

    def wait_gather(b):
        pltpu.make_async_copy(
            table_hbm.at[idx2d.at[0]], bufs[b], sems[b]).wait()

    def store(row, b):
        pltpu.sync_copy(bufs[b], out_hbm.at[base + row])

    for b in range(_NBUF):
        gather(b, b)

    def body(i, _):
        r0 = i * _NBUF
        for b in range(_NBUF):
            r = r0 + b
            wait_gather(b)
            store(r, b)
            gather(r + _NBUF, b)
        return _

    lax.fori_loop(0, (_RPW - _NBUF) // _NBUF, body, None)

    for b in range(_NBUF):
        r = _RPW - _NBUF + b
        wait_gather(b)
        store(r, b)


@functools.partial(
    pl.kernel,
    mesh=plsc.VectorSubcoreMesh(core_axis_name="c", subcore_axis_name="s"),
    out_type=jax.ShapeDtypeStruct((_ROWS, _HIST, _FEAT), jnp.float32),
    scratch_types=[
        pltpu.VMEM((_RPW, _HIST), jnp.int32),
    ] + [pltpu.VMEM((_HIST, _FEAT), jnp.float32) for _ in range(_NBUF)]
      + [pltpu.SemaphoreType.DMA for _ in range(_NBUF)],
    compiler_params=pltpu.CompilerParams(use_tc_tiling_on_sc=False),
)
def _embed_gather(table_hbm, idx_hbm, out_hbm, idx2d, *rest):
    bufs = rest[:_NBUF]
    sems = rest[_NBUF:]
    _embed_body(table_hbm, idx_hbm, out_hbm, idx2d, bufs, sems)


def kernel(metadata, embedding):
    m = metadata
    if m.ndim > 0 and m.shape[-1] == 1:
        m = jnp.squeeze(m, axis=-1)
    return _embed_gather(embedding, m)


# speedup vs baseline: 1.7965x; 1.0013x over previous
import functools

import jax
import jax.numpy as jnp
from jax import lax
from jax.experimental import pallas as pl
from jax.experimental.pallas import tpu as pltpu
from jax.experimental.pallas import tpu_sc as plsc

_FEAT = 32
_ROWS = 16384
_HIST = 50
_NW = 32
_RPW = _ROWS // _NW
_NBUF = 16


def _embed_body(table_hbm, idx_hbm, out_hbm, idx2d, bufs, sems):
    wid = lax.axis_index("s") * 2 + lax.axis_index("c")
    base = wid * _RPW
    pltpu.sync_copy(idx_hbm.at[pl.ds(base, _RPW)], idx2d)

    def gather(row, b):
        pltpu.async_copy(table_hbm.at[idx2d.at[row]], bufs[b], sems[b])

    def wait_gather(b):
        pltpu.make_async_copy(
            table_hbm.at[idx2d.at[0]], bufs[b], sems[b]).wait()

    def store(row, b):
        pltpu.sync_copy(bufs[b], out_hbm.at[base + row])

    for b in range(_NBUF):
        gather(b, b)

    def body(i, _):
        r0 = i * _NBUF
        for b in range(_NBUF):
            r = r0 + b
            wait_gather(b)
            store(r, b)
            gather(r + _NBUF, b)
        return _

    lax.fori_loop(0, (_RPW - _NBUF) // _NBUF, body, None)

    for b in range(_NBUF):
        r = _RPW - _NBUF + b
        wait_gather(b)
        store(r, b)


@functools.partial(
    pl.kernel,
    mesh=plsc.VectorSubcoreMesh(core_axis_name="c", subcore_axis_name="s"),
    out_type=jax.ShapeDtypeStruct((_ROWS, _HIST, _FEAT), jnp.float32),
    scratch_types=[
        pltpu.VMEM((_RPW, _HIST), jnp.int32),
    ] + [pltpu.VMEM((_HIST, _FEAT), jnp.float32) for _ in range(_NBUF)]
      + [pltpu.SemaphoreType.DMA for _ in range(_NBUF)],
    compiler_params=pltpu.CompilerParams(use_tc_tiling_on_sc=False),
)
def _embed_gather(table_hbm, idx_hbm, out_hbm, idx2d, *rest):
    bufs = rest[:_NBUF]
    sems = rest[_NBUF:]
    _embed_body(table_hbm, idx_hbm, out_hbm, idx2d, bufs, sems)


def kernel(metadata, embedding):
    m = metadata
    if m.ndim > 0 and m.shape[-1] == 1:
        m = jnp.squeeze(m, axis=-1)
    return _embed_gather(embedding, m)
